# baseline (speedup 1.0000x reference)
import jax
import jax.numpy as jnp
from jax.experimental import pallas as pl
from jax.experimental.pallas import tpu as pltpu

P = 512
E = 32
H = 64
N = P * P
N2 = N // 2
BLK = 2048


def _sigmoid(x):
    return 0.5 * jnp.tanh(0.5 * x) + 0.5


def _lstm_block(corr_ref, ht_ref, ct_ref, nei_ref,
                w_emb_ref, b_emb_ref, w_i_ref, w_f_ref, w_g_ref, w_o_ref,
                b_i_ref, b_f_ref, b_g_ref, b_o_ref, mask_ones_ref,
                ht_out_ref, ct_out_ref):
    corr = corr_ref[...]
    ht = ht_ref[...]
    ct = ct_ref[...]
    nei = nei_ref[...]

    dn = (((1,), (0,)), ((), ()))

    emb = jnp.maximum(
        jax.lax.dot_general(corr, w_emb_ref[...], dn,
                            preferred_element_type=jnp.float32)
        + b_emb_ref[...], 0.0)

    embh = emb.astype(jnp.bfloat16)
    hth = ht.astype(jnp.bfloat16)

    def gate(w_ref, b_ref):
        w = w_ref[...]
        return (jax.lax.dot_general(embh, w[:2 * E, :], dn,
                                    preferred_element_type=jnp.float32)
                + jax.lax.dot_general(hth, w[2 * E:, :], dn,
                                      preferred_element_type=jnp.float32)
                + b_ref[...])

    i = _sigmoid(gate(w_i_ref, b_i_ref))
    f = _sigmoid(gate(w_f_ref, b_f_ref))
    g = jnp.tanh(gate(w_g_ref, b_g_ref))
    o = _sigmoid(gate(w_o_ref, b_o_ref))

    c_new = f * ct + i * g
    h_new = o * jnp.tanh(c_new)

    mf = jax.lax.dot_general(nei, mask_ones_ref[...], dn,
                             preferred_element_type=jnp.float32)
    ht_out_ref[...] = ht + mf * (h_new - ht)
    ct_out_ref[...] = ct + mf * (c_new - ct)


def _blockdiag2(w):
    k, m = w.shape
    z = jnp.zeros((k, m), w.dtype)
    return jnp.concatenate(
        [jnp.concatenate([w, z], axis=1), jnp.concatenate([z, w], axis=1)],
        axis=0)


def kernel(corr_index, rela_ht, rela_ct, nei_index, W_emb, b_emb, W_ih, W_hh, b_ih, b_hh):
    corr = corr_index.reshape(N2, 4)
    ht = rela_ht.reshape(N2, 2 * H)
    ct = rela_ct.reshape(N2, 2 * H)
    neif = (nei_index.reshape(N2, 2) > 0).astype(jnp.bfloat16)

    w_emb2 = _blockdiag2(W_emb.T)
    b_emb2 = jnp.tile(b_emb, 2).reshape(1, 2 * E)
    w_ih = W_ih.T.astype(jnp.bfloat16)
    w_hh = W_hh.T.astype(jnp.bfloat16)
    bias = (b_ih + b_hh)
    w_gates = []
    b_gates = []
    for k in range(4):
        wi2 = _blockdiag2(w_ih[:, k * H:(k + 1) * H])
        wh2 = _blockdiag2(w_hh[:, k * H:(k + 1) * H])
        w_gates.append(jnp.concatenate([wi2, wh2], axis=0))
        b_gates.append(jnp.tile(bias[k * H:(k + 1) * H], 2).reshape(1, 2 * H))
    mask_ones = _blockdiag2(jnp.ones((1, H), dtype=jnp.bfloat16))

    grid = (N2 // BLK,)
    row_spec = lambda w: pl.BlockSpec((BLK, w), lambda i: (i, 0))
    full_spec = lambda a, b: pl.BlockSpec((a, b), lambda i: (0, 0))

    ht_out, ct_out = pl.pallas_call(
        _lstm_block,
        grid=grid,
        in_specs=[
            row_spec(4),
            row_spec(2 * H),
            row_spec(2 * H),
            row_spec(2),
            full_spec(4, 2 * E),
            full_spec(1, 2 * E),
            full_spec(2 * E + 2 * H, 2 * H),
            full_spec(2 * E + 2 * H, 2 * H),
            full_spec(2 * E + 2 * H, 2 * H),
            full_spec(2 * E + 2 * H, 2 * H),
            full_spec(1, 2 * H),
            full_spec(1, 2 * H),
            full_spec(1, 2 * H),
            full_spec(1, 2 * H),
            full_spec(2, 2 * H),
        ],
        out_specs=[row_spec(2 * H), row_spec(2 * H)],
        out_shape=[
            jax.ShapeDtypeStruct((N2, 2 * H), jnp.float32),
            jax.ShapeDtypeStruct((N2, 2 * H), jnp.float32),
        ],
        compiler_params=pltpu.CompilerParams(
            dimension_semantics=("arbitrary",),
        ),
    )(corr, ht, ct, neif, w_emb2, b_emb2, *w_gates, *b_gates, mask_ones)

    return ht_out.reshape(P, P, H), ct_out.reshape(P, P, H)


# speedup vs baseline: 1.2531x; 1.2531x over previous
import jax
import jax.numpy as jnp
from jax.experimental import pallas as pl
from jax.experimental.pallas import tpu as pltpu

P = 512
E = 32
H = 64
N = P * P
BP = 8
BLK = BP * P


def _sigmoid(x):
    return 0.5 * jnp.tanh(0.5 * x) + 0.5


def _lstm_block(corr_ref, ht_ref, ct_ref, nei_ref,
                w_emb_ref, b_emb_ref, w_i_ref, w_f_ref, w_g_ref, w_o_ref,
                b_i_ref, b_f_ref, b_g_ref, b_o_ref, ones_ref,
                ht_out_ref, ct_out_ref):
    corr = corr_ref[...]
    ht = ht_ref[...].reshape(BLK, H)
    ct = ct_ref[...].reshape(BLK, H)
    nei = nei_ref[...]

    dn = (((1,), (0,)), ((), ()))

    emb = jnp.maximum(
        jax.lax.dot_general(corr, w_emb_ref[...], dn,
                            preferred_element_type=jnp.float32)
        + b_emb_ref[...], 0.0)

    embh = emb.astype(jnp.bfloat16)
    hth = ht.astype(jnp.bfloat16)

    def gate(w_ref, b_ref):
        w = w_ref[...]
        return (jax.lax.dot_general(embh, w[:E, :], dn,
                                    preferred_element_type=jnp.float32)
                + jax.lax.dot_general(hth, w[E:, :], dn,
                                      preferred_element_type=jnp.float32)
                + b_ref[...])

    i = _sigmoid(gate(w_i_ref, b_i_ref))
    f = _sigmoid(gate(w_f_ref, b_f_ref))
    g = jnp.tanh(gate(w_g_ref, b_g_ref))
    o = _sigmoid(gate(w_o_ref, b_o_ref))

    c_new = f * ct + i * g
    h_new = o * jnp.tanh(c_new)

    mf = jax.lax.dot_general(nei, ones_ref[...], dn,
                             preferred_element_type=jnp.float32)
    ht_out_ref[...] = (ht + mf * (h_new - ht)).reshape(BP, P, H)
    ct_out_ref[...] = (ct + mf * (c_new - ct)).reshape(BP, P, H)


def kernel(corr_index, rela_ht, rela_ct, nei_index, W_emb, b_emb, W_ih, W_hh, b_ih, b_hh):
    corr = corr_index.reshape(N, 2)
    neif = (nei_index.reshape(N, 1) > 0).astype(jnp.bfloat16)

    w_emb = W_emb.T
    b_emb_r = b_emb.reshape(1, E)
    w_ih = W_ih.T.astype(jnp.bfloat16)
    w_hh = W_hh.T.astype(jnp.bfloat16)
    bias = (b_ih + b_hh).reshape(1, 4 * H)
    w_gates = [jnp.concatenate([w_ih[:, k * H:(k + 1) * H],
                                w_hh[:, k * H:(k + 1) * H]], axis=0)
               for k in range(4)]
    b_gates = [bias[:, k * H:(k + 1) * H] for k in range(4)]
    ones_h = jnp.ones((1, H), dtype=jnp.bfloat16)

    grid = (P // BP,)
    state_spec = pl.BlockSpec((BP, P, H), lambda i: (i, 0, 0))
    row_spec = lambda w: pl.BlockSpec((BLK, w), lambda i: (i, 0))
    full_spec = lambda a, b: pl.BlockSpec((a, b), lambda i: (0, 0))

    ht_out, ct_out = pl.pallas_call(
        _lstm_block,
        grid=grid,
        in_specs=[
            row_spec(2),
            state_spec,
            state_spec,
            row_spec(1),
            full_spec(2, E),
            full_spec(1, E),
            full_spec(E + H, H),
            full_spec(E + H, H),
            full_spec(E + H, H),
            full_spec(E + H, H),
            full_spec(1, H),
            full_spec(1, H),
            full_spec(1, H),
            full_spec(1, H),
            full_spec(1, H),
        ],
        out_specs=[state_spec, state_spec],
        out_shape=[
            jax.ShapeDtypeStruct((P, P, H), jnp.float32),
            jax.ShapeDtypeStruct((P, P, H), jnp.float32),
        ],
        compiler_params=pltpu.CompilerParams(
            dimension_semantics=("arbitrary",),
        ),
    )(corr, rela_ht, rela_ct, neif, w_emb, b_emb_r, *w_gates, *b_gates, ones_h)

    return ht_out, ct_out


# speedup vs baseline: 5.4560x; 4.3539x over previous
import jax
import jax.numpy as jnp
from jax.experimental import pallas as pl
from jax.experimental.pallas import tpu as pltpu

P = 512
E = 32
H = 64
BP = 8


def _sigmoid(x):
    return 0.5 * jnp.tanh(0.5 * x) + 0.5


def _lstm_block(corr_ref, ht_ref, ct_ref, nei_ref,
                w_emb_ref, w_ih_ref, w_hh_ref, b_col_ref, ones_ref,
                ht_out_ref, ct_out_ref):
    w_emb = w_emb_ref[...]
    w_ih = w_ih_ref[...]
    w_hh = w_hh_ref[...]
    dn = (((1,), (0,)), ((), ()))

    bias = jax.lax.dot_general(b_col_ref[...], ones_ref[...], dn,
                               preferred_element_type=jnp.float32)

    for p in range(BP):
        corr = corr_ref[3 * p:3 * p + 3, :]
        ht = ht_ref[H * p:H * p + H, :]
        ct = ct_ref[H * p:H * p + H, :]
        m = nei_ref[p:p + 1, :] > 0

        emb = jnp.maximum(
            jax.lax.dot_general(w_emb, corr, dn,
                                preferred_element_type=jnp.float32), 0.0)

        gates = (jax.lax.dot_general(w_ih, emb.astype(jnp.bfloat16), dn,
                                     preferred_element_type=jnp.float32)
                 + jax.lax.dot_general(w_hh, ht.astype(jnp.bfloat16), dn,
                                       preferred_element_type=jnp.float32)
                 + bias)

        i = _sigmoid(gates[0:H, :])
        f = _sigmoid(gates[H:2 * H, :])
        g = jnp.tanh(gates[2 * H:3 * H, :])
        o = _sigmoid(gates[3 * H:4 * H, :])

        c_new = f * ct + i * g
        h_new = o * jnp.tanh(c_new)

        ht_out_ref[H * p:H * p + H, :] = jnp.where(m, h_new, ht)
        ct_out_ref[H * p:H * p + H, :] = jnp.where(m, c_new, ct)


def kernel(corr_index, rela_ht, rela_ct, nei_index, W_emb, b_emb, W_ih, W_hh, b_ih, b_hh):
    ht = rela_ht.transpose(0, 2, 1).reshape(P * H, P)
    ct = rela_ct.transpose(0, 2, 1).reshape(P * H, P)
    corr_t = corr_index.transpose(0, 2, 1)
    ones_row = jnp.ones((P, 1, P), jnp.float32)
    corr_aug = jnp.concatenate([corr_t, ones_row], axis=1).reshape(P * 3, P)
    nei = nei_index.astype(jnp.int32)

    w_emb_aug = jnp.concatenate([W_emb, b_emb[:, None]], axis=1)
    w_ih = W_ih.astype(jnp.bfloat16)
    w_hh = W_hh.astype(jnp.bfloat16)
    b_col = (b_ih + b_hh)[:, None]
    ones_p = jnp.ones((1, P), jnp.float32)

    grid = (P // BP,)
    ht_out, ct_out = pl.pallas_call(
        _lstm_block,
        grid=grid,
        in_specs=[
            pl.BlockSpec((3 * BP, P), lambda i: (i, 0)),
            pl.BlockSpec((H * BP, P), lambda i: (i, 0)),
            pl.BlockSpec((H * BP, P), lambda i: (i, 0)),
            pl.BlockSpec((BP, P), lambda i: (i, 0)),
            pl.BlockSpec((E, 3), lambda i: (0, 0)),
            pl.BlockSpec((4 * H, E), lambda i: (0, 0)),
            pl.BlockSpec((4 * H, H), lambda i: (0, 0)),
            pl.BlockSpec((4 * H, 1), lambda i: (0, 0)),
            pl.BlockSpec((1, P), lambda i: (0, 0)),
        ],
        out_specs=[
            pl.BlockSpec((H * BP, P), lambda i: (i, 0)),
            pl.BlockSpec((H * BP, P), lambda i: (i, 0)),
        ],
        out_shape=[
            jax.ShapeDtypeStruct((P * H, P), jnp.float32),
            jax.ShapeDtypeStruct((P * H, P), jnp.float32),
        ],
        compiler_params=pltpu.CompilerParams(
            dimension_semantics=("arbitrary",),
        ),
    )(corr_aug, ht, ct, nei, w_emb_aug, w_ih, w_hh, b_col, ones_p)

    return (ht_out.reshape(P, H, P).transpose(0, 2, 1),
            ct_out.reshape(P, H, P).transpose(0, 2, 1))


# speedup vs baseline: 5.8062x; 1.0642x over previous
import jax
import jax.numpy as jnp
from jax.experimental import pallas as pl
from jax.experimental.pallas import tpu as pltpu

P = 512
E = 32
H = 64
BP = 8


def _lstm_block(corr_ref, ht_ref, ct_ref, nei_ref,
                w_emb_ref, w_ih_ref, w_hh_ref,
                ht_out_ref, ct_out_ref):
    w_emb = w_emb_ref[...]
    w_ih = w_ih_ref[...]
    w_hh = w_hh_ref[...]
    dn = (((1,), (0,)), ((), ()))

    for p in range(BP):
        corr = corr_ref[3 * p:3 * p + 3, :]
        ht = ht_ref[H * p:H * p + H, :]
        ct = ct_ref[H * p:H * p + H, :]
        m = nei_ref[p:p + 1, :] > 0

        emb = jnp.maximum(
            jax.lax.dot_general(w_emb, corr, dn,
                                preferred_element_type=jnp.float32),
            0.0)

        gates = (jax.lax.dot_general(w_ih, emb.astype(jnp.bfloat16), dn,
                                     preferred_element_type=jnp.float32)
                 + jax.lax.dot_general(w_hh, ht.astype(jnp.bfloat16), dn,
                                       preferred_element_type=jnp.float32))

        i = 0.5 * jnp.tanh(gates[0:H, :]) + 0.5
        f = 0.5 * jnp.tanh(gates[H:2 * H, :]) + 0.5
        g = jnp.tanh(gates[2 * H:3 * H, :])
        o = 0.5 * jnp.tanh(gates[3 * H:4 * H, :]) + 0.5

        c_new = f * ct + i * g
        h_new = o * jnp.tanh(c_new)

        ht_out_ref[H * p:H * p + H, :] = jnp.where(m, h_new, ht)
        ct_out_ref[H * p:H * p + H, :] = jnp.where(m, c_new, ct)


def kernel(corr_index, rela_ht, rela_ct, nei_index, W_emb, b_emb, W_ih, W_hh, b_ih, b_hh):
    ht = rela_ht.transpose(0, 2, 1).reshape(P * H, P)
    ct = rela_ct.transpose(0, 2, 1).reshape(P * H, P)
    corr_t = corr_index.transpose(0, 2, 1)
    ones_row = jnp.ones((P, 1, P), jnp.float32)
    corr_aug = jnp.concatenate([corr_t, ones_row], axis=1)
    corr_aug = corr_aug.reshape(P * 3, P).astype(jnp.bfloat16)
    nei = nei_index.astype(jnp.int32)

    w_emb_aug = jnp.concatenate([W_emb, b_emb[:, None]], axis=1)
    w_emb_aug = jnp.concatenate(
        [w_emb_aug, jnp.array([[0.0, 0.0, 1.0]], jnp.float32)], axis=0)

    bias = (b_ih + b_hh)[:, None]
    w_ih_aug = jnp.concatenate([W_ih, bias], axis=1)
    scale = jnp.where((jnp.arange(4 * H) < 2 * H) | (jnp.arange(4 * H) >= 3 * H),
                      0.5, 1.0)[:, None]
    w_ih_aug = (w_ih_aug * scale).astype(jnp.bfloat16)
    w_hh_s = (W_hh * scale).astype(jnp.bfloat16)

    grid = (P // BP,)
    ht_out, ct_out = pl.pallas_call(
        _lstm_block,
        grid=grid,
        in_specs=[
            pl.BlockSpec((3 * BP, P), lambda i: (i, 0)),
            pl.BlockSpec((H * BP, P), lambda i: (i, 0)),
            pl.BlockSpec((H * BP, P), lambda i: (i, 0)),
            pl.BlockSpec((BP, P), lambda i: (i, 0)),
            pl.BlockSpec((E + 1, 3), lambda i: (0, 0)),
            pl.BlockSpec((4 * H, E + 1), lambda i: (0, 0)),
            pl.BlockSpec((4 * H, H), lambda i: (0, 0)),
        ],
        out_specs=[
            pl.BlockSpec((H * BP, P), lambda i: (i, 0)),
            pl.BlockSpec((H * BP, P), lambda i: (i, 0)),
        ],
        out_shape=[
            jax.ShapeDtypeStruct((P * H, P), jnp.float32),
            jax.ShapeDtypeStruct((P * H, P), jnp.float32),
        ],
        compiler_params=pltpu.CompilerParams(
            dimension_semantics=("arbitrary",),
        ),
    )(corr_aug, ht, ct, nei, w_emb_aug.astype(jnp.bfloat16), w_ih_aug, w_hh_s)

    return (ht_out.reshape(P, H, P).transpose(0, 2, 1),
            ct_out.reshape(P, H, P).transpose(0, 2, 1))


# speedup vs baseline: 6.2754x; 1.0808x over previous
import jax
import jax.numpy as jnp
from jax.experimental import pallas as pl
from jax.experimental.pallas import tpu as pltpu

P = 512
E = 32
H = 64
BP = 16


def _lstm_block(corr_ref, ht_ref, ct_ref, nei_ref,
                w_emb_ref, w_ih_ref, w_hh_ref,
                ht_out_ref, ct_out_ref):
    w_emb = w_emb_ref[...]
    w_ih = w_ih_ref[...]
    w_hh = w_hh_ref[...]
    dn = (((1,), (0,)), ((), ()))

    for p in range(BP):
        corr = corr_ref[3 * p:3 * p + 3, :]
        ht = ht_ref[H * p:H * p + H, :]
        ct = ct_ref[H * p:H * p + H, :]
        m = nei_ref[p:p + 1, :] > 0

        emb = jnp.maximum(
            jax.lax.dot_general(w_emb, corr, dn,
                                preferred_element_type=jnp.float32),
            0.0)

        gates = (jax.lax.dot_general(w_ih, emb.astype(jnp.bfloat16), dn,
                                     preferred_element_type=jnp.float32)
                 + jax.lax.dot_general(w_hh, ht.astype(jnp.bfloat16), dn,
                                       preferred_element_type=jnp.float32))

        i = 0.5 * jnp.tanh(gates[0:H, :]) + 0.5
        f = 0.5 * jnp.tanh(gates[H:2 * H, :]) + 0.5
        g = jnp.tanh(gates[2 * H:3 * H, :])
        o = 0.5 * jnp.tanh(gates[3 * H:4 * H, :]) + 0.5

        c_new = f * ct + i * g
        h_new = o * jnp.tanh(c_new)

        ht_out_ref[H * p:H * p + H, :] = jnp.where(m, h_new, ht)
        ct_out_ref[H * p:H * p + H, :] = jnp.where(m, c_new, ct)


def kernel(corr_index, rela_ht, rela_ct, nei_index, W_emb, b_emb, W_ih, W_hh, b_ih, b_hh):
    ht = rela_ht.transpose(0, 2, 1).reshape(P * H, P)
    ct = rela_ct.transpose(0, 2, 1).reshape(P * H, P)
    corr_t = corr_index.transpose(0, 2, 1)
    ones_row = jnp.ones((P, 1, P), jnp.float32)
    corr_aug = jnp.concatenate([corr_t, ones_row], axis=1)
    corr_aug = corr_aug.reshape(P * 3, P).astype(jnp.bfloat16)
    nei = nei_index.astype(jnp.int32)

    w_emb_aug = jnp.concatenate([W_emb, b_emb[:, None]], axis=1)
    w_emb_aug = jnp.concatenate(
        [w_emb_aug, jnp.array([[0.0, 0.0, 1.0]], jnp.float32)], axis=0)

    bias = (b_ih + b_hh)[:, None]
    w_ih_aug = jnp.concatenate([W_ih, bias], axis=1)
    scale = jnp.where((jnp.arange(4 * H) < 2 * H) | (jnp.arange(4 * H) >= 3 * H),
                      0.5, 1.0)[:, None]
    w_ih_aug = (w_ih_aug * scale).astype(jnp.bfloat16)
    w_hh_s = (W_hh * scale).astype(jnp.bfloat16)

    grid = (P // BP,)
    ht_out, ct_out = pl.pallas_call(
        _lstm_block,
        grid=grid,
        in_specs=[
            pl.BlockSpec((3 * BP, P), lambda i: (i, 0)),
            pl.BlockSpec((H * BP, P), lambda i: (i, 0)),
            pl.BlockSpec((H * BP, P), lambda i: (i, 0)),
            pl.BlockSpec((BP, P), lambda i: (i, 0)),
            pl.BlockSpec((E + 1, 3), lambda i: (0, 0)),
            pl.BlockSpec((4 * H, E + 1), lambda i: (0, 0)),
            pl.BlockSpec((4 * H, H), lambda i: (0, 0)),
        ],
        out_specs=[
            pl.BlockSpec((H * BP, P), lambda i: (i, 0)),
            pl.BlockSpec((H * BP, P), lambda i: (i, 0)),
        ],
        out_shape=[
            jax.ShapeDtypeStruct((P * H, P), jnp.float32),
            jax.ShapeDtypeStruct((P * H, P), jnp.float32),
        ],
        compiler_params=pltpu.CompilerParams(
            dimension_semantics=("arbitrary",),
        ),
    )(corr_aug, ht, ct, nei, w_emb_aug.astype(jnp.bfloat16), w_ih_aug, w_hh_s)

    return (ht_out.reshape(P, H, P).transpose(0, 2, 1),
            ct_out.reshape(P, H, P).transpose(0, 2, 1))
